# Initial kernel scaffold; baseline (speedup 1.0000x reference)
import jax
import jax.numpy as jnp
from jax.experimental import pallas as pl


def kernel(center_map):
    raise NotImplementedError("write your pallas kernel here")


# speedup vs baseline: 7.5424x; 7.5424x over previous
import functools

import jax
import jax.numpy as jnp
from jax import lax
from jax.experimental import pallas as pl
from jax.experimental.pallas import tpu as pltpu
from jax.experimental.pallas import tpu_sc as plsc

_THR = 0.1
_TOPK = 200
_B, _H, _W = 16, 512, 512
_IMG = _H * _W
_NW = 32
_HALF = _IMG // 2
_CAP = 4096
_CHUNK = 16384
_LANES = 16


def _nms_body(x_ref, out_ref):
    x = x_ref[0]
    neg = jnp.float32(-jnp.inf)
    m = x
    for s in (1, 2, 3):
        lo = jnp.concatenate([x[:, s:], jnp.full((_H, s), neg, x.dtype)], axis=1)
        hi = jnp.concatenate([jnp.full((_H, s), neg, x.dtype), x[:, :-s]], axis=1)
        m = jnp.maximum(m, jnp.maximum(lo, hi))
    p = m
    for s in (1, 2, 3):
        lo = jnp.concatenate([m[s:, :], jnp.full((s, _W), neg, m.dtype)], axis=0)
        hi = jnp.concatenate([jnp.full((s, _W), neg, m.dtype), m[:-s, :]], axis=0)
        p = jnp.maximum(p, jnp.maximum(lo, hi))
    keep = (p > _THR) & (p == x)
    out_ref[0] = jnp.where(keep, x, 0.0)


_nms_call = pl.pallas_call(
    _nms_body,
    out_shape=jax.ShapeDtypeStruct((_B, _H, _W), jnp.float32),
    grid=(_B,),
    in_specs=[pl.BlockSpec((1, _H, _W), lambda i: (i, 0, 0))],
    out_specs=pl.BlockSpec((1, _H, _W), lambda i: (i, 0, 0)),
)


@functools.cache
def _make_compact():
    mesh = plsc.VectorSubcoreMesh(core_axis_name="c", subcore_axis_name="s")
    return pl.kernel(
        _compact_body,
        out_type=(
            jax.ShapeDtypeStruct((_NW, _CAP), jnp.float32),
            jax.ShapeDtypeStruct((_NW, _CAP), jnp.int32),
        ),
        mesh=mesh,
        scratch_types=[
            pltpu.VMEM((_CHUNK,), jnp.float32),
            pltpu.VMEM((_CAP,), jnp.float32),
            pltpu.VMEM((_CAP,), jnp.int32),
            pltpu.VMEM((_LANES,), jnp.int32),
        ],
        compiler_params=pltpu.CompilerParams(needs_layout_passes=False),
    )


def _compact_body(nms_hbm, ovals_hbm, oidx_hbm, buf, cvals, cidx, cntbuf):
    wid = lax.axis_index("s") * 2 + lax.axis_index("c")
    base = wid * _HALF
    local_base = (wid % 2) * _HALF

    zf = jnp.zeros((_LANES,), jnp.float32)
    zi = jnp.zeros((_LANES,), jnp.int32)

    def zero_body(j, carry):
        cvals[pl.ds(j * _LANES, _LANES)] = zf
        cidx[pl.ds(j * _LANES, _LANES)] = zi
        return carry

    lax.fori_loop(0, _CAP // _LANES, zero_body, 0)

    lane = lax.iota(jnp.int32, _LANES)

    def chunk_body(c, off):
        pltpu.sync_copy(nms_hbm.at[pl.ds(base + c * _CHUNK, _CHUNK)], buf)

        def vec_body(j, off):
            v = buf[pl.ds(j * _LANES, _LANES)]
            msk = v > 0.0
            idxv = (local_base + c * _CHUNK + j * _LANES) + lane
            cnt = plsc.all_reduce_population_count(msk)[0]
            slot = jnp.minimum(off, _CAP - _LANES)
            plsc.store_compressed(cvals.at[pl.ds(slot, _LANES)], v, mask=msk)
            plsc.store_compressed(cidx.at[pl.ds(slot, _LANES)], idxv, mask=msk)
            return off + cnt

        return lax.fori_loop(0, _CHUNK // _LANES, vec_body, off)

    lax.fori_loop(0, _HALF // _CHUNK, chunk_body, jnp.int32(0))
    pltpu.sync_copy(cvals, ovals_hbm.at[wid])
    pltpu.sync_copy(cidx, oidx_hbm.at[wid])


def _select_body(vals_ref, idx_ref, coords_ref, probs_ref, v_scr, i_acc):
    v_scr[...] = vals_ref[...]
    big = jnp.int32(1 << 30)
    col = lax.broadcasted_iota(jnp.int32, (_B, _TOPK), 1)

    def body(r, carry):
        v = v_scr[...]
        m = jnp.max(v, axis=1, keepdims=True)
        tie = v == m
        idxs = idx_ref[...]
        imin = jnp.min(jnp.where(tie, idxs, big), axis=1, keepdims=True)
        hit = col == r
        probs_ref[...] = jnp.where(hit, m, probs_ref[...])
        i_acc[...] = jnp.where(hit, imin, i_acc[...])
        v_scr[...] = jnp.where(tie & (idxs == imin), jnp.float32(-1.0), v)
        return carry

    lax.fori_loop(0, _TOPK, body, 0)
    packed = i_acc[...]
    coords_ref[0] = packed // _W
    coords_ref[1] = packed % _W


_select_call = pl.pallas_call(
    _select_body,
    out_shape=(
        jax.ShapeDtypeStruct((2, _B, _TOPK), jnp.int32),
        jax.ShapeDtypeStruct((_B, _TOPK), jnp.float32),
    ),
    scratch_shapes=[
        pltpu.VMEM((_B, 2 * _CAP), jnp.float32),
        pltpu.VMEM((_B, _TOPK), jnp.int32),
    ],
)


@jax.jit
def kernel(center_map):
    x = center_map.reshape(_B, _H, _W)
    nms = _nms_call(x)
    vals, idx = _make_compact()(nms.reshape(_B * _IMG))
    coords2, probs = _select_call(
        vals.reshape(_B, 2 * _CAP), idx.reshape(_B, 2 * _CAP)
    )
    coords = jnp.stack([coords2[0], coords2[1]], axis=-1)
    return coords, probs


# speedup vs baseline: 8.3757x; 1.1105x over previous
import functools

import jax
import jax.numpy as jnp
from jax import lax
from jax.experimental import pallas as pl
from jax.experimental.pallas import tpu as pltpu
from jax.experimental.pallas import tpu_sc as plsc

_THR = 0.1
_TOPK = 200
_B, _H, _W = 16, 512, 512
_IMG = _H * _W
_NW = 32
_HALF = _IMG // 2
_CAP = 4096
_CHUNK = 16384
_LANES = 16


def _win7_lanes(x):
    neg = jnp.float32(-jnp.inf)

    def shl(a, s):
        return jnp.concatenate([a[:, s:], jnp.full((_H, s), neg, a.dtype)], axis=1)

    def shr(a, s):
        return jnp.concatenate([jnp.full((_H, s), neg, a.dtype), a[:, :-s]], axis=1)

    g1 = jnp.maximum(x, shl(x, 1))
    g2 = jnp.maximum(g1, shl(g1, 2))
    r1 = jnp.maximum(x, shr(x, 1))
    r2 = jnp.maximum(r1, shr(r1, 2))
    return jnp.maximum(g2, r2)


def _win7_sublanes(x):
    neg = jnp.float32(-jnp.inf)

    def shu(a, s):
        return jnp.concatenate([a[s:, :], jnp.full((s, _W), neg, a.dtype)], axis=0)

    def shd(a, s):
        return jnp.concatenate([jnp.full((s, _W), neg, a.dtype), a[:-s, :]], axis=0)

    g1 = jnp.maximum(x, shu(x, 1))
    g2 = jnp.maximum(g1, shu(g1, 2))
    r1 = jnp.maximum(x, shd(x, 1))
    r2 = jnp.maximum(r1, shd(r1, 2))
    return jnp.maximum(g2, r2)


def _nms_body(x_ref, out_ref):
    x = x_ref[0]
    p = _win7_sublanes(_win7_lanes(x))
    keep = (p > _THR) & (p == x)
    out_ref[0] = jnp.where(keep, x, 0.0)


_nms_call = pl.pallas_call(
    _nms_body,
    out_shape=jax.ShapeDtypeStruct((_B, _H, _W), jnp.float32),
    grid=(_B,),
    in_specs=[pl.BlockSpec((1, _H, _W), lambda i: (i, 0, 0))],
    out_specs=pl.BlockSpec((1, _H, _W), lambda i: (i, 0, 0)),
)


@functools.cache
def _make_compact():
    mesh = plsc.VectorSubcoreMesh(core_axis_name="c", subcore_axis_name="s")
    return pl.kernel(
        _compact_body,
        out_type=(
            jax.ShapeDtypeStruct((_B, 2 * _CAP), jnp.float32),
            jax.ShapeDtypeStruct((_B, 2 * _CAP), jnp.int32),
        ),
        mesh=mesh,
        scratch_types=[
            pltpu.VMEM((_CHUNK,), jnp.float32),
            pltpu.VMEM((_CHUNK,), jnp.float32),
            pltpu.VMEM((_CAP,), jnp.float32),
            pltpu.VMEM((_CAP,), jnp.int32),
            pltpu.SemaphoreType.DMA,
            pltpu.SemaphoreType.DMA,
        ],
        compiler_params=pltpu.CompilerParams(needs_layout_passes=False),
    )


def _compact_body(nms_hbm, ovals_hbm, oidx_hbm, buf0, buf1, cvals, cidx, sem0, sem1):
    wid = lax.axis_index("s") * 2 + lax.axis_index("c")
    img = wid // 2
    half = wid % 2
    base = wid * _HALF
    local_base = half * _HALF

    zf = jnp.zeros((_LANES,), jnp.float32)
    zi = jnp.zeros((_LANES,), jnp.int32)

    def zero_body(j, carry):
        cvals[pl.ds(j * _LANES, _LANES)] = zf
        cidx[pl.ds(j * _LANES, _LANES)] = zi
        return carry

    lax.fori_loop(0, _CAP // _LANES, zero_body, 0, unroll=4)

    lane = lax.iota(jnp.int32, _LANES)
    bufs = (buf0, buf1)
    sems = (sem0, sem1)
    n_chunks = _HALF // _CHUNK

    def start(c):
        return pltpu.async_copy(
            nms_hbm.at[pl.ds(base + c * _CHUNK, _CHUNK)], bufs[c % 2], sems[c % 2]
        )

    pending = start(0)
    off = jnp.int32(0)
    for c in range(n_chunks):
        nxt = start(c + 1) if c + 1 < n_chunks else None
        pending.wait()
        cbuf = bufs[c % 2]
        cbase = local_base + c * _CHUNK

        def vec_body(j, off, cbuf=cbuf, cbase=cbase):
            v = cbuf[pl.ds(j * _LANES, _LANES)]
            msk = v > 0.0
            idxv = (cbase + j * _LANES) + lane
            cnt = plsc.all_reduce_population_count(msk)[0]
            slot = jnp.minimum(off, _CAP - _LANES)
            plsc.store_compressed(cvals.at[pl.ds(slot, _LANES)], v, mask=msk)
            plsc.store_compressed(cidx.at[pl.ds(slot, _LANES)], idxv, mask=msk)
            return off + cnt

        off = lax.fori_loop(0, _CHUNK // _LANES, vec_body, off, unroll=8)
        pending = nxt

    pltpu.sync_copy(cvals, ovals_hbm.at[img, pl.ds(half * _CAP, _CAP)])
    pltpu.sync_copy(cidx, oidx_hbm.at[img, pl.ds(half * _CAP, _CAP)])


def _select_body(vals_ref, idx_ref, coords_ref, probs_ref, v_scr, i_acc):
    v_scr[...] = vals_ref[...]
    big = jnp.int32(1 << 30)
    col = lax.broadcasted_iota(jnp.int32, (_B, _TOPK), 1)

    def body(r, carry):
        v = v_scr[...]
        m = jnp.max(v, axis=1, keepdims=True)
        tie = v == m
        idxs = idx_ref[...]
        imin = jnp.min(jnp.where(tie, idxs, big), axis=1, keepdims=True)
        hit = col == r
        probs_ref[...] = jnp.where(hit, m, probs_ref[...])
        i_acc[...] = jnp.where(hit, imin, i_acc[...])
        v_scr[...] = jnp.where(tie & (idxs == imin), jnp.float32(-1.0), v)
        return carry

    lax.fori_loop(0, _TOPK, body, 0)
    packed = i_acc[...]
    coords_ref[0] = packed // _W
    coords_ref[1] = packed % _W


_select_call = pl.pallas_call(
    _select_body,
    out_shape=(
        jax.ShapeDtypeStruct((2, _B, _TOPK), jnp.int32),
        jax.ShapeDtypeStruct((_B, _TOPK), jnp.float32),
    ),
    scratch_shapes=[
        pltpu.VMEM((_B, 2 * _CAP), jnp.float32),
        pltpu.VMEM((_B, _TOPK), jnp.int32),
    ],
)


@jax.jit
def kernel(center_map):
    x = center_map.reshape(_B, _H, _W)
    nms = _nms_call(x)
    vals, idx = _make_compact()(nms.reshape(_B * _IMG))
    coords2, probs = _select_call(vals, idx)
    coords = jnp.stack([coords2[0], coords2[1]], axis=-1)
    return coords, probs
